# Initial kernel scaffold; baseline (speedup 1.0000x reference)
import jax
import jax.numpy as jnp
from jax.experimental import pallas as pl


def kernel(x, params):
    raise NotImplementedError("write your pallas kernel here")


# speedup vs baseline: 3.2915x; 3.2915x over previous
import functools

import jax
import jax.numpy as jnp
from jax import lax
from jax.experimental import pallas as pl
from jax.experimental.pallas import tpu as pltpu
from jax.experimental.pallas import tpu_sc as plsc

B, C0, N, KNN = 4, 3, 2048, 20
CP = 64
GW = 128
RB = 256
NT = 256
EPS = 1e-5
ECNT = float(B * KNN * N)


def _leaky(x):
    return jnp.where(x >= 0, x, 0.2 * x)


def _knn_body(hf_ref, hb_ref, xxf_ref, xxb_ref, idx_ref, cv_ref, pd_ref):
    b = pl.program_id(0)
    hf = hf_ref[0]
    hb = hb_ref[0]
    cv_ref[0] = jnp.concatenate([hb[:, :CP], -hb[:, :CP]], axis=1)
    dmat = lax.dot_general(hb, hf, (((1,), (1,)), ((), ())),
                           preferred_element_type=jnp.float32)
    inner = -2.0 * dmat
    xxb = xxb_ref[0][:, 0:1]
    xxf = xxf_ref[0, 0]
    t = jnp.negative(xxb) - inner
    pd_ref[...] = t - xxf[None, :]
    iota = lax.broadcasted_iota(jnp.int32, (RB, N), 1)
    base = b * N

    def step(t, carry):
        sc = pd_ref[...]
        m = jnp.max(sc, axis=1)
        cand = jnp.where(sc == m[:, None], iota, N)
        j = jnp.min(cand, axis=1)
        idx_ref[0, pl.ds(t, 1), :] = (j + base)[None, :]
        pd_ref[...] = jnp.where(iota == j[:, None], -jnp.inf, sc)
        return carry

    lax.fori_loop(0, KNN, step, 0)


def _knn(h, xx3, xxb):
    grid = (B, N // RB)
    return pl.pallas_call(
        _knn_body,
        grid=grid,
        in_specs=[
            pl.BlockSpec((1, N, GW), lambda b, n: (b, 0, 0)),
            pl.BlockSpec((1, RB, GW), lambda b, n: (b, n, 0)),
            pl.BlockSpec((1, 1, N), lambda b, n: (b, 0, 0)),
            pl.BlockSpec((1, RB, 128), lambda b, n: (b, n, 0)),
        ],
        out_specs=[
            pl.BlockSpec((1, KNN, RB), lambda b, n: (b, 0, n)),
            pl.BlockSpec((1, RB, GW), lambda b, n: (b, n, 0)),
        ],
        out_shape=[
            jax.ShapeDtypeStruct((B, KNN, N), jnp.int32),
            jax.ShapeDtypeStruct((B, N, GW), jnp.float32),
        ],
        scratch_shapes=[pltpu.VMEM((RB, N), jnp.float32)],
    )(h, h, xx3, xxb)


def _gather_rows(table, idx):
    info = plsc.get_sparse_core_info()
    nw = info.num_cores * info.num_subcores
    tot = idx.shape[0]
    per_w = tot // nw
    ch = 128
    n_ch = per_w // ch
    mesh = plsc.VectorSubcoreMesh(core_axis_name="c", subcore_axis_name="s")

    @functools.partial(
        pl.kernel,
        mesh=mesh,
        out_type=jax.ShapeDtypeStruct((tot, GW), jnp.float32),
        scratch_types=[
            pltpu.VMEM((ch,), jnp.int32),
            pltpu.VMEM((ch, GW), jnp.float32),
            pltpu.SemaphoreType.DMA,
        ],
    )
    def k(table_hbm, idx_hbm, out_hbm, idx_v, rows_v, sem):
        wid = lax.axis_index("s") * info.num_cores + lax.axis_index("c")
        base = wid * per_w

        def body(c, carry):
            off = base + c * ch
            pltpu.sync_copy(idx_hbm.at[pl.ds(off, ch)], idx_v)
            pltpu.async_copy(table_hbm.at[idx_v], rows_v, sem).wait()
            pltpu.sync_copy(rows_v, out_hbm.at[pl.ds(off, ch)])
            return carry

        lax.fori_loop(0, n_ch, body, 0)

    return k(table, idx)


def _conv1_body(g_ref, cv_ref, w_ref, y_ref):
    e = g_ref[0, 0] - cv_ref[0]
    y_ref[0, 0] = jnp.dot(e, w_ref[...], preferred_element_type=jnp.float32)


def _conv1(g, cv, w1p):
    grid = (B, KNN)
    return pl.pallas_call(
        _conv1_body,
        grid=grid,
        in_specs=[
            pl.BlockSpec((1, 1, N, GW), lambda b, k: (b, k, 0, 0)),
            pl.BlockSpec((1, N, GW), lambda b, k: (b, 0, 0)),
            pl.BlockSpec((GW, CP), lambda b, k: (0, 0)),
        ],
        out_specs=pl.BlockSpec((1, 1, N, CP), lambda b, k: (b, k, 0, 0)),
        out_shape=jax.ShapeDtypeStruct((B, KNN, N, CP), jnp.float32),
    )(g, cv, w1p)


def _bn_mirror(y, mu_row, v_row, g_row, b_row):
    xh = (y - mu_row) / jnp.sqrt(v_row + EPS)
    return _leaky(xh * g_row + b_row)


def _conv2_body(y1_ref, mu_ref, v_ref, g1_ref, b1_ref, w_ref, y_ref):
    z1 = _bn_mirror(y1_ref[0, 0], mu_ref[0:1, :], v_ref[0:1, :],
                    g1_ref[...], b1_ref[...])
    y_ref[0, 0] = jnp.dot(z1, w_ref[...], preferred_element_type=jnp.float32)


def _conv2(y1, mu1, v1, g1, b1, w2t):
    grid = (B, KNN)
    return pl.pallas_call(
        _conv2_body,
        grid=grid,
        in_specs=[
            pl.BlockSpec((1, 1, N, CP), lambda b, k: (b, k, 0, 0)),
            pl.BlockSpec((8, CP), lambda b, k: (0, 0)),
            pl.BlockSpec((8, CP), lambda b, k: (0, 0)),
            pl.BlockSpec((1, CP), lambda b, k: (0, 0)),
            pl.BlockSpec((1, CP), lambda b, k: (0, 0)),
            pl.BlockSpec((CP, CP), lambda b, k: (0, 0)),
        ],
        out_specs=pl.BlockSpec((1, 1, N, CP), lambda b, k: (b, k, 0, 0)),
        out_shape=jax.ShapeDtypeStruct((B, KNN, N, CP), jnp.float32),
    )(y1, mu1, v1, g1, b1, w2t)


def _final_body(y2_ref, mu_ref, v_ref, g2_ref, b2_ref, h_ref):
    k = pl.program_id(1)
    z2 = _bn_mirror(y2_ref[0, 0], mu_ref[0:1, :], v_ref[0:1, :],
                    g2_ref[...], b2_ref[...])
    z2 = jnp.pad(z2, ((0, 0), (0, GW - CP)))

    @pl.when(k == 0)
    def _():
        h_ref[0] = z2

    @pl.when(k > 0)
    def _():
        h_ref[0] = jnp.maximum(h_ref[0], z2)


def _finalize(y2, mu2, v2, g2, b2):
    grid = (B, KNN)
    return pl.pallas_call(
        _final_body,
        grid=grid,
        in_specs=[
            pl.BlockSpec((1, 1, N, CP), lambda b, k: (b, k, 0, 0)),
            pl.BlockSpec((8, CP), lambda b, k: (0, 0)),
            pl.BlockSpec((8, CP), lambda b, k: (0, 0)),
            pl.BlockSpec((1, CP), lambda b, k: (0, 0)),
            pl.BlockSpec((1, CP), lambda b, k: (0, 0)),
        ],
        out_specs=pl.BlockSpec((1, N, GW), lambda b, k: (b, 0, 0)),
        out_shape=jax.ShapeDtypeStruct((B, N, GW), jnp.float32),
    )(y2, mu2, v2, g2, b2)


def _mean_body(x_ref, o_ref, acc_ref):
    i0, i1 = pl.program_id(0), pl.program_id(1)
    first = (i0 == 0) & (i1 == 0)
    last = ((i0 == pl.num_programs(0) - 1) &
            (i1 == pl.num_programs(1) - 1))
    xb = x_ref[0]

    @pl.when(first)
    def _():
        acc_ref[...] = jnp.zeros((CP, 8, KNN), jnp.float32)

    acc_ref[...] += jnp.sum(xb.reshape(CP, NT // 8, 8, KNN), axis=1)

    @pl.when(last)
    def _():
        s = jnp.sum(acc_ref[...], axis=(1, 2))
        o_ref[...] = jnp.broadcast_to((s / ECNT)[None, :], (8, CP))


def _mean_ref_layout(xt):
    grid = (B, N // NT)
    return pl.pallas_call(
        _mean_body,
        grid=grid,
        in_specs=[pl.BlockSpec((1, CP, NT, KNN), lambda b, n: (b, 0, n, 0))],
        out_specs=pl.BlockSpec((8, CP), lambda b, n: (0, 0)),
        out_shape=jax.ShapeDtypeStruct((8, CP), jnp.float32),
        scratch_shapes=[pltpu.VMEM((CP, 8, KNN), jnp.float32)],
    )(xt)


def _var_body(x_ref, mu_ref, o_ref, acc_ref):
    i0, i1 = pl.program_id(0), pl.program_id(1)
    first = (i0 == 0) & (i1 == 0)
    last = ((i0 == pl.num_programs(0) - 1) &
            (i1 == pl.num_programs(1) - 1))
    xc = x_ref[0] - mu_ref[...]
    xc = xc * xc

    @pl.when(first)
    def _():
        acc_ref[...] = jnp.zeros((CP, 8, KNN), jnp.float32)

    acc_ref[...] += jnp.sum(xc.reshape(CP, NT // 8, 8, KNN), axis=1)

    @pl.when(last)
    def _():
        s = jnp.sum(acc_ref[...], axis=(1, 2))
        o_ref[...] = jnp.broadcast_to((s / ECNT)[None, :], (8, CP))


def _var_ref_layout(xt, mu_full):
    grid = (B, N // NT)
    return pl.pallas_call(
        _var_body,
        grid=grid,
        in_specs=[
            pl.BlockSpec((1, CP, NT, KNN), lambda b, n: (b, 0, n, 0)),
            pl.BlockSpec((CP, NT, KNN), lambda b, n: (0, 0, 0)),
        ],
        out_specs=pl.BlockSpec((8, CP), lambda b, n: (0, 0)),
        out_shape=jax.ShapeDtypeStruct((8, CP), jnp.float32),
        scratch_shapes=[pltpu.VMEM((CP, 8, KNN), jnp.float32)],
    )(xt, mu_full)


def _bn_stats(y):
    yt = jnp.transpose(y, (0, 3, 2, 1))
    mu = _mean_ref_layout(yt)
    mu_full = jnp.broadcast_to(mu[0][:, None, None], (CP, NT, KNN))
    v = _var_ref_layout(yt, mu_full)
    return mu, v


def _edge_conv(h, xx, w1, g1v, b1v, w2, g2v, b2v, cin):
    xx3 = xx[:, None, :]
    xxb = jnp.broadcast_to(xx[:, :, None], (B, N, 128))
    idx, cv = _knn(h, xx3, xxb)
    g = _gather_rows(h.reshape(B * N, GW), idx.reshape(B * KNN * N))
    g = g.reshape(B, KNN, N, GW)
    w1p = jnp.zeros((GW, CP), jnp.float32)
    w1p = w1p.at[:cin, :].set(w1[:, :cin].T)
    w1p = w1p.at[CP:CP + cin, :].set(w1[:, cin:].T)
    y1 = _conv1(g, cv, w1p)
    mu1, v1 = _bn_stats(y1)
    g1r, b1r = g1v.reshape(1, CP), b1v.reshape(1, CP)
    g2r, b2r = g2v.reshape(1, CP), b2v.reshape(1, CP)
    y2 = _conv2(y1, mu1, v1, g1r, b1r, w2.T)
    mu2, v2 = _bn_stats(y2)
    return _finalize(y2, mu2, v2, g2r, b2r)


NBM = 512


def _mlp1_body(hc_ref, w_ref, y_ref, s_ref, ss_ref):
    first = (pl.program_id(0) == 0) & (pl.program_id(1) == 0)
    y = jnp.dot(hc_ref[0], w_ref[...], preferred_element_type=jnp.float32)
    y_ref[0] = y
    y4 = y.reshape(8, NBM // 8, 512)
    s8 = jnp.sum(y4, axis=1)
    ss8 = jnp.sum(y4 * y4, axis=1)

    @pl.when(first)
    def _():
        s_ref[...] = s8
        ss_ref[...] = ss8

    @pl.when(jnp.logical_not(first))
    def _():
        s_ref[...] += s8
        ss_ref[...] += ss8


def _mlp1(hc, w0t):
    grid = (B, N // NBM)
    return pl.pallas_call(
        _mlp1_body,
        grid=grid,
        in_specs=[
            pl.BlockSpec((1, NBM, 192), lambda b, n: (b, n, 0)),
            pl.BlockSpec((192, 512), lambda b, n: (0, 0)),
        ],
        out_specs=[
            pl.BlockSpec((1, NBM, 512), lambda b, n: (b, n, 0)),
            pl.BlockSpec((8, 512), lambda b, n: (0, 0)),
            pl.BlockSpec((8, 512), lambda b, n: (0, 0)),
        ],
        out_shape=[
            jax.ShapeDtypeStruct((B, N, 512), jnp.float32),
            jax.ShapeDtypeStruct((8, 512), jnp.float32),
            jax.ShapeDtypeStruct((8, 512), jnp.float32),
        ],
    )(hc, w0t)


def _mlp2_body(y0_ref, s0_ref, ss0_ref, g0_ref, b0_ref, w_ref,
               y_ref, s_ref, ss_ref):
    first = (pl.program_id(0) == 0) & (pl.program_id(1) == 0)
    cnt = float(B * N)
    s0 = jnp.sum(s0_ref[...], axis=0, keepdims=True)
    ss0 = jnp.sum(ss0_ref[...], axis=0, keepdims=True)
    mu = s0 / cnt
    var = ss0 / cnt - mu * mu
    z0 = _bn_mirror(y0_ref[0], mu, var, g0_ref[...], b0_ref[...])
    y = jnp.dot(z0, w_ref[...], preferred_element_type=jnp.float32)
    y_ref[0] = y
    y4 = y.reshape(8, NBM // 8, 256)
    s8 = jnp.sum(y4, axis=1)
    ss8 = jnp.sum(y4 * y4, axis=1)

    @pl.when(first)
    def _():
        s_ref[...] = s8
        ss_ref[...] = ss8

    @pl.when(jnp.logical_not(first))
    def _():
        s_ref[...] += s8
        ss_ref[...] += ss8


def _mlp2(y0, s0, ss0, g0, b0, w1t):
    grid = (B, N // NBM)
    return pl.pallas_call(
        _mlp2_body,
        grid=grid,
        in_specs=[
            pl.BlockSpec((1, NBM, 512), lambda b, n: (b, n, 0)),
            pl.BlockSpec((8, 512), lambda b, n: (0, 0)),
            pl.BlockSpec((8, 512), lambda b, n: (0, 0)),
            pl.BlockSpec((1, 512), lambda b, n: (0, 0)),
            pl.BlockSpec((1, 512), lambda b, n: (0, 0)),
            pl.BlockSpec((512, 256), lambda b, n: (0, 0)),
        ],
        out_specs=[
            pl.BlockSpec((1, NBM, 256), lambda b, n: (b, n, 0)),
            pl.BlockSpec((8, 256), lambda b, n: (0, 0)),
            pl.BlockSpec((8, 256), lambda b, n: (0, 0)),
        ],
        out_shape=[
            jax.ShapeDtypeStruct((B, N, 256), jnp.float32),
            jax.ShapeDtypeStruct((8, 256), jnp.float32),
            jax.ShapeDtypeStruct((8, 256), jnp.float32),
        ],
    )(y0, s0, ss0, g0, b0, w1t)


def _qkv_body(y1_ref, s1_ref, ss1_ref, g1_ref, b1_ref,
              wq_ref, bq_ref, wk_ref, bk_ref, wv_ref, bv_ref,
              o_ref, q_ref, k_ref, v_ref):
    cnt = float(B * N)
    s1 = jnp.sum(s1_ref[...], axis=0, keepdims=True)
    ss1 = jnp.sum(ss1_ref[...], axis=0, keepdims=True)
    mu = s1 / cnt
    var = ss1 / cnt - mu * mu
    om = _bn_mirror(y1_ref[0], mu, var, g1_ref[...], b1_ref[...])
    o_ref[0] = om
    q_ref[0] = jnp.dot(om, wq_ref[...],
                       preferred_element_type=jnp.float32) + bq_ref[...]
    k_ref[0] = jnp.dot(om, wk_ref[...],
                       preferred_element_type=jnp.float32) + bk_ref[...]
    v_ref[0] = jnp.dot(om, wv_ref[...],
                       preferred_element_type=jnp.float32) + bv_ref[...]


def _qkv(y1, s1, ss1, g1, b1, wqt, bq, wkt, bk, wvt, bv):
    grid = (B, N // NBM)
    return pl.pallas_call(
        _qkv_body,
        grid=grid,
        in_specs=[
            pl.BlockSpec((1, NBM, 256), lambda b, n: (b, n, 0)),
            pl.BlockSpec((8, 256), lambda b, n: (0, 0)),
            pl.BlockSpec((8, 256), lambda b, n: (0, 0)),
            pl.BlockSpec((1, 256), lambda b, n: (0, 0)),
            pl.BlockSpec((1, 256), lambda b, n: (0, 0)),
            pl.BlockSpec((256, 128), lambda b, n: (0, 0)),
            pl.BlockSpec((1, 128), lambda b, n: (0, 0)),
            pl.BlockSpec((256, 128), lambda b, n: (0, 0)),
            pl.BlockSpec((1, 128), lambda b, n: (0, 0)),
            pl.BlockSpec((256, 256), lambda b, n: (0, 0)),
            pl.BlockSpec((1, 256), lambda b, n: (0, 0)),
        ],
        out_specs=[
            pl.BlockSpec((1, NBM, 256), lambda b, n: (b, n, 0)),
            pl.BlockSpec((1, NBM, 128), lambda b, n: (b, n, 0)),
            pl.BlockSpec((1, NBM, 128), lambda b, n: (b, n, 0)),
            pl.BlockSpec((1, NBM, 256), lambda b, n: (b, n, 0)),
        ],
        out_shape=[
            jax.ShapeDtypeStruct((B, N, 256), jnp.float32),
            jax.ShapeDtypeStruct((B, N, 128), jnp.float32),
            jax.ShapeDtypeStruct((B, N, 128), jnp.float32),
            jax.ShapeDtypeStruct((B, N, 256), jnp.float32),
        ],
    )(y1, s1, ss1, g1, b1, wqt, bq, wkt, bk, wvt, bv)


def _attn_body(q_ref, k_ref, v_ref, o_ref, gm_ref, out_ref):
    s = lax.dot_general(q_ref[0], k_ref[0], (((1,), (1,)), ((), ())),
                        preferred_element_type=jnp.float32)
    m = jnp.max(s, axis=1)
    p = jnp.exp(s - m[:, None])
    den = jnp.sum(p, axis=1)
    attn = p / den[:, None]
    ao = jnp.dot(attn, v_ref[0], preferred_element_type=jnp.float32)
    out_ref[0] = gm_ref[0, 0] * ao + o_ref[0]


def _attention(q, k, v, om, gamma):
    grid = (B, N // RB)
    return pl.pallas_call(
        _attn_body,
        grid=grid,
        in_specs=[
            pl.BlockSpec((1, RB, 128), lambda b, n: (b, n, 0)),
            pl.BlockSpec((1, N, 128), lambda b, n: (b, 0, 0)),
            pl.BlockSpec((1, N, 256), lambda b, n: (b, 0, 0)),
            pl.BlockSpec((1, RB, 256), lambda b, n: (b, n, 0)),
            pl.BlockSpec((1, 1), lambda b, n: (0, 0)),
        ],
        out_specs=pl.BlockSpec((1, RB, 256), lambda b, n: (b, n, 0)),
        out_shape=jax.ShapeDtypeStruct((B, N, 256), jnp.float32),
    )(q, k, v, om, gamma)


def kernel(x, params):
    p = params
    h = jnp.transpose(x, (0, 2, 1))
    h = jnp.pad(h, ((0, 0), (0, 0), (0, GW - C0)))
    feats = []
    cins = [C0, CP, CP]
    for i in range(3):
        if i == 0:
            xx = jnp.sum(x * x, axis=1)
        else:
            hT = jnp.transpose(h[:, :, :CP], (0, 2, 1))
            xx = jnp.sum(hT * hT, axis=1)
        h = _edge_conv(h, xx, p['ec%d_w0' % i], p['ec%d_g0' % i],
                       p['ec%d_b0' % i], p['ec%d_w1' % i],
                       p['ec%d_g1' % i], p['ec%d_b1' % i], cins[i])
        feats.append(h)
    hc = jnp.concatenate([f[:, :, :CP] for f in feats], axis=2)
    y0, s0, ss0 = _mlp1(hc, p['mlp_w0'].T)
    g0, b0 = p['mlp_g0'].reshape(1, 512), p['mlp_b0'].reshape(1, 512)
    y1, s1, ss1 = _mlp2(y0, s0, ss0, g0, b0, p['mlp_w1'].T)
    g1, b1 = p['mlp_g1'].reshape(1, 256), p['mlp_b1'].reshape(1, 256)
    om, q, k, v = _qkv(y1, s1, ss1, g1, b1,
                       p['wq'].T, p['bq'].reshape(1, 128),
                       p['wk'].T, p['bk'].reshape(1, 128),
                       p['wv'].T, p['bv'].reshape(1, 256))
    out2 = _attention(q, k, v, om, p['gamma'].reshape(1, 1))
    feat0 = jnp.transpose(feats[0][:, :, :CP], (0, 2, 1))
    out2 = jnp.transpose(out2, (0, 2, 1))
    return (feat0, out2)
